# baseline (speedup 1.0000x reference)
import functools

import jax
import jax.numpy as jnp
from jax import lax
from jax.experimental import pallas as pl
from jax.experimental.pallas import tpu as pltpu
from jax.experimental.pallas import tpu_sc as plsc

N = 10000
E = 320000
NC, NS, L = 2, 16, 16
NW = NC * NS
CHUNK = 128
PT = 10240
K = PT // CHUNK
EP = NW * PT
NP = 10240
SLC = NP // NS
NBUF = 2
NO = K // NBUF
UNROLL = 4

_mesh = plsc.VectorSubcoreMesh(core_axis_name="c", subcore_axis_name="s")


def _sc_degree(dst3):
    @functools.partial(
        pl.kernel,
        out_type=jax.ShapeDtypeStruct((NC, NP), jnp.float32),
        mesh=_mesh,
        scratch_types=[
            pltpu.VMEM((K, CHUNK), jnp.int32),
            pltpu.VMEM((CHUNK,), jnp.float32),
            pltpu.VMEM((SLC,), jnp.float32),
            pltpu.VMEM_SHARED((NP,), jnp.float32),
        ],
    )
    def deg_kernel(dst_hbm, deg_hbm, dst_v, ones_v, zer_v, acc):
        c = lax.axis_index("c")
        s = lax.axis_index("s")
        w = c * NS + s
        pltpu.sync_copy(dst_hbm.at[w], dst_v)
        for i in range(CHUNK // L):
            ones_v[pl.ds(i * L, L)] = jnp.ones((L,), jnp.float32)
        for i in range(SLC // L):
            zer_v[pl.ds(i * L, L)] = jnp.zeros((L,), jnp.float32)
        pltpu.sync_copy(zer_v, acc.at[pl.ds(s * SLC, SLC)])
        plsc.subcore_barrier()

        def body(o, carry):
            for i in range(UNROLL):
                j = o * UNROLL + i
                pltpu.sync_copy(ones_v, acc.at[dst_v.at[j]], add=True)
            return carry

        lax.fori_loop(0, K // UNROLL, body, 0)
        plsc.subcore_barrier()
        pltpu.sync_copy(acc.at[pl.ds(s * SLC, SLC)],
                        deg_hbm.at[c, pl.ds(s * SLC, SLC)])

    return deg_kernel(dst3)


@functools.lru_cache(maxsize=None)
def _make_sc_aggregate(F):
    @functools.partial(
        pl.kernel,
        out_type=jax.ShapeDtypeStruct((NC, NP, F), jnp.float32),
        mesh=_mesh,
        scratch_types=[
            pltpu.VMEM((NBUF, CHUNK), jnp.int32),
            pltpu.VMEM((NBUF, CHUNK), jnp.int32),
            pltpu.VMEM((NBUF, CHUNK, F), jnp.float32),
            pltpu.VMEM((L, F), jnp.float32),
            pltpu.VMEM_SHARED((NP, F), jnp.float32),
        ] + [pltpu.SemaphoreType.DMA] * (4 * NBUF),
    )
    def agg_kernel(g_hbm, src_hbm, dst_hbm, out_hbm,
                   src_v, dst_v, rows_v, zer_v, acc, *sems):
        sem_g = sems[0 * NBUF:1 * NBUF]
        sem_s = sems[1 * NBUF:2 * NBUF]
        sem_i = sems[2 * NBUF:3 * NBUF]
        sem_d = sems[3 * NBUF:4 * NBUF]
        c = lax.axis_index("c")
        s = lax.axis_index("s")
        w = c * NS + s
        for i in range(L):
            for k in range(F // L):
                zer_v[i, pl.ds(k * L, L)] = jnp.zeros((L,), jnp.float32)
        for t in range(SLC // L):
            pltpu.sync_copy(zer_v, acc.at[pl.ds(s * SLC + t * L, L)])
        plsc.subcore_barrier()

        def load_idx(j, b):
            pltpu.async_copy(src_hbm.at[w, j], src_v.at[b], sem_i[b])
            pltpu.async_copy(dst_hbm.at[w, j], dst_v.at[b], sem_d[b])

        def wait_idx(b):
            pltpu.make_async_copy(src_hbm.at[w, 0], src_v.at[b], sem_i[b]).wait()
            pltpu.make_async_copy(dst_hbm.at[w, 0], dst_v.at[b], sem_d[b]).wait()

        def gather(b):
            pltpu.async_copy(g_hbm.at[src_v.at[b]], rows_v.at[b], sem_g[b])

        def wait_gather(b):
            pltpu.make_async_copy(
                g_hbm.at[src_v.at[b]], rows_v.at[b], sem_g[b]).wait()

        def scatter(b):
            pltpu.async_copy(rows_v.at[b], acc.at[dst_v.at[b]], sem_s[b],
                             add=True)

        def wait_scatter(b):
            pltpu.make_async_copy(
                rows_v.at[b], acc.at[dst_v.at[b]], sem_s[b]).wait()

        for b in range(NBUF):
            load_idx(b, b)
        for b in range(NBUF):
            wait_idx(b)
            gather(b)

        def body(o, carry):
            for b in range(NBUF):
                j = o * NBUF + b
                wait_gather(b)
                scatter(b)
                wait_scatter(b)
                load_idx(lax.rem(j + NBUF, K), b)
                wait_idx(b)
                gather(b)
            return carry

        lax.fori_loop(0, NO, body, 0)
        for b in range(NBUF):
            wait_gather(b)
        plsc.subcore_barrier()
        pltpu.sync_copy(acc.at[pl.ds(s * SLC, SLC)],
                        out_hbm.at[c, pl.ds(s * SLC, SLC)])

    return agg_kernel


def _sc_aggregate(g, src3, dst3, F):
    return _make_sc_aggregate(F)(g, src3, dst3)


_R = 2000


def _tc_first(x, W1, deg0, deg1):
    def body(x_ref, w_ref, d0_ref, d1_ref, g_ref, dinv_ref):
        dinv = lax.rsqrt(d0_ref[...] + d1_ref[...] + 1.0)
        dinv_ref[...] = dinv
        h = jnp.dot(x_ref[...], w_ref[...], preferred_element_type=jnp.float32)
        g_ref[...] = h * dinv

    F = W1.shape[1]
    return pl.pallas_call(
        body,
        grid=(N // _R,),
        in_specs=[
            pl.BlockSpec((_R, x.shape[1]), lambda i: (i, 0)),
            pl.BlockSpec(W1.shape, lambda i: (0, 0)),
            pl.BlockSpec((_R, 1), lambda i: (i, 0)),
            pl.BlockSpec((_R, 1), lambda i: (i, 0)),
        ],
        out_specs=[
            pl.BlockSpec((_R, F), lambda i: (i, 0)),
            pl.BlockSpec((_R, 1), lambda i: (i, 0)),
        ],
        out_shape=[
            jax.ShapeDtypeStruct((N, F), jnp.float32),
            jax.ShapeDtypeStruct((N, 1), jnp.float32),
        ],
    )(x, W1, deg0, deg1)


def _tc_mid(a, g, dinv, b, Wn):
    def body(a0_ref, a1_ref, g_ref, dinv_ref, b_ref, w_ref, out_ref):
        dinv = dinv_ref[...]
        h = (a0_ref[0] + a1_ref[0] + g_ref[...]) * dinv + b_ref[...]
        h = jnp.maximum(h, 0.0)
        out_ref[...] = jnp.dot(
            h, w_ref[...], preferred_element_type=jnp.float32) * dinv

    F = g.shape[1]
    Fn = Wn.shape[1]
    return pl.pallas_call(
        body,
        grid=(N // _R,),
        in_specs=[
            pl.BlockSpec((1, _R, F), lambda i: (0, i, 0)),
            pl.BlockSpec((1, _R, F), lambda i: (1, i, 0)),
            pl.BlockSpec((_R, F), lambda i: (i, 0)),
            pl.BlockSpec((_R, 1), lambda i: (i, 0)),
            pl.BlockSpec((1, F), lambda i: (0, 0)),
            pl.BlockSpec(Wn.shape, lambda i: (0, 0)),
        ],
        out_specs=pl.BlockSpec((_R, Fn), lambda i: (i, 0)),
        out_shape=jax.ShapeDtypeStruct((N, Fn), jnp.float32),
    )(a, a, g, dinv, b, Wn)


def _tc_last(a, g, dinv, b):
    def body(a0_ref, a1_ref, g_ref, dinv_ref, b_ref, out_ref):
        v = (a0_ref[0] + a1_ref[0] + g_ref[...]) * dinv_ref[...]
        v = v[:, :64] + b_ref[...]
        m = jnp.max(v, axis=-1, keepdims=True)
        z = v - m
        out_ref[...] = z - jnp.log(jnp.sum(jnp.exp(z), axis=-1, keepdims=True))

    F = g.shape[1]
    return pl.pallas_call(
        body,
        grid=(N // _R,),
        in_specs=[
            pl.BlockSpec((1, _R, F), lambda i: (0, i, 0)),
            pl.BlockSpec((1, _R, F), lambda i: (1, i, 0)),
            pl.BlockSpec((_R, F), lambda i: (i, 0)),
            pl.BlockSpec((_R, 1), lambda i: (i, 0)),
            pl.BlockSpec((1, 64), lambda i: (0, 0)),
        ],
        out_specs=pl.BlockSpec((_R, 64), lambda i: (i, 0)),
        out_shape=jax.ShapeDtypeStruct((N, 64), jnp.float32),
    )(a, a, g, dinv, b)


def kernel(x, edge_index, W1, b1, W2, b2, W3, b3):
    src = edge_index[0].astype(jnp.int32)
    dst = edge_index[1].astype(jnp.int32)
    pad = EP - E
    src3 = jnp.concatenate([src, jnp.zeros((pad,), jnp.int32)]).reshape(NW, K, CHUNK)
    dst3 = jnp.concatenate([dst, jnp.full((pad,), N, jnp.int32)]).reshape(NW, K, CHUNK)

    deg = _sc_degree(dst3)
    deg0 = deg[0, :N].reshape(N, 1)
    deg1 = deg[1, :N].reshape(N, 1)

    g1, dinv = _tc_first(x, W1, deg0, deg1)
    a1 = _sc_aggregate(g1, src3, dst3, 128)
    g2 = _tc_mid(a1, g1, dinv, b1.reshape(1, -1), W2)
    a2 = _sc_aggregate(g2, src3, dst3, 128)
    W3p = jnp.pad(W3, ((0, 0), (0, 128 - W3.shape[1])))
    g3 = _tc_mid(a2, g2, dinv, b2.reshape(1, -1), W3p)
    a3 = _sc_aggregate(g3, src3, dst3, 128)
    return _tc_last(a3, g3, dinv, b3.reshape(1, -1))


# speedup vs baseline: 1.0141x; 1.0141x over previous
import functools

import jax
import jax.numpy as jnp
from jax import lax
from jax.experimental import pallas as pl
from jax.experimental.pallas import tpu as pltpu
from jax.experimental.pallas import tpu_sc as plsc

N = 10000
E = 320000
NC, NS, L = 2, 16, 16
NW = NC * NS
CHUNK = 128
PT = 10240
K = PT // CHUNK
EP = NW * PT
NP = 10240
SLC = NP // NS
NBUF = 2
NO = K // NBUF
UNROLL = 4

_mesh = plsc.VectorSubcoreMesh(core_axis_name="c", subcore_axis_name="s")


def _sc_degree(dst3):
    @functools.partial(
        pl.kernel,
        out_type=jax.ShapeDtypeStruct((NC, NP), jnp.float32),
        mesh=_mesh,
        scratch_types=[
            pltpu.VMEM((K, CHUNK), jnp.int32),
            pltpu.VMEM((CHUNK,), jnp.float32),
            pltpu.VMEM((SLC,), jnp.float32),
            pltpu.VMEM_SHARED((NP,), jnp.float32),
        ],
    )
    def deg_kernel(dst_hbm, deg_hbm, dst_v, ones_v, zer_v, acc):
        c = lax.axis_index("c")
        s = lax.axis_index("s")
        w = c * NS + s
        pltpu.sync_copy(dst_hbm.at[w], dst_v)
        for i in range(CHUNK // L):
            ones_v[pl.ds(i * L, L)] = jnp.ones((L,), jnp.float32)
        for i in range(SLC // L):
            zer_v[pl.ds(i * L, L)] = jnp.zeros((L,), jnp.float32)
        pltpu.sync_copy(zer_v, acc.at[pl.ds(s * SLC, SLC)])
        plsc.subcore_barrier()

        def body(o, carry):
            for i in range(UNROLL):
                j = o * UNROLL + i
                pltpu.sync_copy(ones_v, acc.at[dst_v.at[j]], add=True)
            return carry

        lax.fori_loop(0, K // UNROLL, body, 0)
        plsc.subcore_barrier()
        pltpu.sync_copy(acc.at[pl.ds(s * SLC, SLC)],
                        deg_hbm.at[c, pl.ds(s * SLC, SLC)])

    return deg_kernel(dst3)


@functools.lru_cache(maxsize=None)
def _make_sc_aggregate(F):
    @functools.partial(
        pl.kernel,
        out_type=jax.ShapeDtypeStruct((NC, NP, F), jnp.float32),
        mesh=_mesh,
        scratch_types=[
            pltpu.VMEM((NBUF, CHUNK), jnp.int32),
            pltpu.VMEM((NBUF, CHUNK), jnp.int32),
            pltpu.VMEM((NBUF, CHUNK, F), jnp.float32),
            pltpu.VMEM((L, F), jnp.float32),
            pltpu.VMEM_SHARED((NP, F), jnp.float32),
        ] + [pltpu.SemaphoreType.DMA] * (4 * NBUF + 1),
    )
    def agg_kernel(g_hbm, src_hbm, dst_hbm, out_hbm,
                   src_v, dst_v, rows_v, zer_v, acc, *sems):
        sem_g = sems[0 * NBUF:1 * NBUF]
        sem_s = sems[1 * NBUF:2 * NBUF]
        sem_i = sems[2 * NBUF:3 * NBUF]
        sem_d = sems[3 * NBUF:4 * NBUF]
        sem_z = sems[4 * NBUF]
        c = lax.axis_index("c")
        s = lax.axis_index("s")
        w = c * NS + s
        for i in range(L):
            for k in range(F // L):
                zer_v[i, pl.ds(k * L, L)] = jnp.zeros((L,), jnp.float32)

        def load_idx(j, b):
            pltpu.async_copy(src_hbm.at[w, j], src_v.at[b], sem_i[b])
            pltpu.async_copy(dst_hbm.at[w, j], dst_v.at[b], sem_d[b])

        def wait_idx(b):
            pltpu.make_async_copy(src_hbm.at[w, 0], src_v.at[b], sem_i[b]).wait()
            pltpu.make_async_copy(dst_hbm.at[w, 0], dst_v.at[b], sem_d[b]).wait()

        def gather(b):
            pltpu.async_copy(g_hbm.at[src_v.at[b]], rows_v.at[b], sem_g[b])

        def wait_gather(b):
            pltpu.make_async_copy(
                g_hbm.at[src_v.at[b]], rows_v.at[b], sem_g[b]).wait()

        def scatter(b):
            pltpu.async_copy(rows_v.at[b], acc.at[dst_v.at[b]], sem_s[b],
                             add=True)

        def wait_scatter(b):
            pltpu.make_async_copy(
                rows_v.at[b], acc.at[dst_v.at[b]], sem_s[b]).wait()

        for b in range(NBUF):
            load_idx(b, b)
        for b in range(NBUF):
            wait_idx(b)
            gather(b)
        for t in range(SLC // L):
            pltpu.async_copy(zer_v, acc.at[pl.ds(s * SLC + t * L, L)], sem_z)
        for t in range(SLC // L):
            pltpu.make_async_copy(
                zer_v, acc.at[pl.ds(s * SLC, L)], sem_z).wait()
        plsc.subcore_barrier()

        def body(o, carry):
            for b in range(NBUF):
                j = o * NBUF + b
                wait_gather(b)
                scatter(b)
                wait_scatter(b)

                @pl.when(j + NBUF < K)
                def _prefetch():
                    load_idx(j + NBUF, b)
                    wait_idx(b)
                    gather(b)

            return carry

        lax.fori_loop(0, NO, body, 0)
        plsc.subcore_barrier()
        pltpu.sync_copy(acc.at[pl.ds(s * SLC, SLC)],
                        out_hbm.at[c, pl.ds(s * SLC, SLC)])

    return agg_kernel


def _sc_aggregate(g, src3, dst3, F):
    return _make_sc_aggregate(F)(g, src3, dst3)


_R = 2000


def _tc_first(x, W1, deg0, deg1):
    def body(x_ref, w_ref, d0_ref, d1_ref, g_ref, dinv_ref):
        dinv = lax.rsqrt(d0_ref[...] + d1_ref[...] + 1.0)
        dinv_ref[...] = dinv
        h = jnp.dot(x_ref[...], w_ref[...], preferred_element_type=jnp.float32)
        g_ref[...] = h * dinv

    F = W1.shape[1]
    return pl.pallas_call(
        body,
        grid=(N // _R,),
        in_specs=[
            pl.BlockSpec((_R, x.shape[1]), lambda i: (i, 0)),
            pl.BlockSpec(W1.shape, lambda i: (0, 0)),
            pl.BlockSpec((_R, 1), lambda i: (i, 0)),
            pl.BlockSpec((_R, 1), lambda i: (i, 0)),
        ],
        out_specs=[
            pl.BlockSpec((_R, F), lambda i: (i, 0)),
            pl.BlockSpec((_R, 1), lambda i: (i, 0)),
        ],
        out_shape=[
            jax.ShapeDtypeStruct((N, F), jnp.float32),
            jax.ShapeDtypeStruct((N, 1), jnp.float32),
        ],
    )(x, W1, deg0, deg1)


def _tc_mid(a, g, dinv, b, Wn):
    def body(a0_ref, a1_ref, g_ref, dinv_ref, b_ref, w_ref, out_ref):
        dinv = dinv_ref[...]
        h = (a0_ref[0] + a1_ref[0] + g_ref[...]) * dinv + b_ref[...]
        h = jnp.maximum(h, 0.0)
        out_ref[...] = jnp.dot(
            h, w_ref[...], preferred_element_type=jnp.float32) * dinv

    F = g.shape[1]
    Fn = Wn.shape[1]
    return pl.pallas_call(
        body,
        grid=(N // _R,),
        in_specs=[
            pl.BlockSpec((1, _R, F), lambda i: (0, i, 0)),
            pl.BlockSpec((1, _R, F), lambda i: (1, i, 0)),
            pl.BlockSpec((_R, F), lambda i: (i, 0)),
            pl.BlockSpec((_R, 1), lambda i: (i, 0)),
            pl.BlockSpec((1, F), lambda i: (0, 0)),
            pl.BlockSpec(Wn.shape, lambda i: (0, 0)),
        ],
        out_specs=pl.BlockSpec((_R, Fn), lambda i: (i, 0)),
        out_shape=jax.ShapeDtypeStruct((N, Fn), jnp.float32),
    )(a, a, g, dinv, b, Wn)


def _tc_last(a, g, dinv, b):
    def body(a0_ref, a1_ref, g_ref, dinv_ref, b_ref, out_ref):
        v = (a0_ref[0] + a1_ref[0] + g_ref[...]) * dinv_ref[...]
        v = v[:, :64] + b_ref[...]
        m = jnp.max(v, axis=-1, keepdims=True)
        z = v - m
        out_ref[...] = z - jnp.log(jnp.sum(jnp.exp(z), axis=-1, keepdims=True))

    F = g.shape[1]
    return pl.pallas_call(
        body,
        grid=(N // _R,),
        in_specs=[
            pl.BlockSpec((1, _R, F), lambda i: (0, i, 0)),
            pl.BlockSpec((1, _R, F), lambda i: (1, i, 0)),
            pl.BlockSpec((_R, F), lambda i: (i, 0)),
            pl.BlockSpec((_R, 1), lambda i: (i, 0)),
            pl.BlockSpec((1, 64), lambda i: (0, 0)),
        ],
        out_specs=pl.BlockSpec((_R, 64), lambda i: (i, 0)),
        out_shape=jax.ShapeDtypeStruct((N, 64), jnp.float32),
    )(a, a, g, dinv, b)


def kernel(x, edge_index, W1, b1, W2, b2, W3, b3):
    src = edge_index[0].astype(jnp.int32)
    dst = edge_index[1].astype(jnp.int32)
    pad = EP - E
    src3 = jnp.concatenate([src, jnp.zeros((pad,), jnp.int32)]).reshape(NW, K, CHUNK)
    dst3 = jnp.concatenate([dst, jnp.full((pad,), N, jnp.int32)]).reshape(NW, K, CHUNK)

    deg = _sc_degree(dst3)
    deg0 = deg[0, :N].reshape(N, 1)
    deg1 = deg[1, :N].reshape(N, 1)

    g1, dinv = _tc_first(x, W1, deg0, deg1)
    a1 = _sc_aggregate(g1, src3, dst3, 128)
    g2 = _tc_mid(a1, g1, dinv, b1.reshape(1, -1), W2)
    a2 = _sc_aggregate(g2, src3, dst3, 128)
    W3p = jnp.pad(W3, ((0, 0), (0, 128 - W3.shape[1])))
    g3 = _tc_mid(a2, g2, dinv, b2.reshape(1, -1), W3p)
    a3 = _sc_aggregate(g3, src3, dst3, 128)
    return _tc_last(a3, g3, dinv, b3.reshape(1, -1))
